# baseline (speedup 1.0000x reference)
import functools

import jax
import jax.numpy as jnp
from jax import lax
from jax.experimental import pallas as pl
from jax.experimental.pallas import tpu as pltpu
from jax.experimental.pallas import tpu_sc as plsc

_EPS = 1e-6
_F, _K = 2048, 8
_NC, _NS, _L = 2, 16, 16
_NW = _NC * _NS
_CB = 128
_RB = 256


def _densify_sc_body(idx_hbm, w1_hbm, w2_hbm, s1_hbm, s2_hbm,
                     strips, idxb, w1b, w2b, sems):
    wid = lax.axis_index("s") * _NC + lax.axis_index("c")
    cb = jnp.right_shift(wid, 1)
    rb0 = jnp.bitwise_and(wid, 1) * 4
    cbase = cb * _CB
    lanes = lax.iota(jnp.int32, _L)
    zeros16 = jnp.zeros((_L,), jnp.float32)

    pltpu.sync_copy(idx_hbm.at[pl.ds(cbase * _K, _CB * _K)], idxb)
    pltpu.sync_copy(w1_hbm.at[pl.ds(cbase * _K, _CB * _K)], w1b)
    pltpu.sync_copy(w2_hbm.at[pl.ds(cbase * _K, _CB * _K)], w2b)
    rows_l, cols_l, v1_l, v2_l = [], [], [], []
    for k in range(_K):
        for half in range(_CB // _L):
            fv = lanes + half * _L
            gi = fv * _K + k
            rows_l.append(plsc.load_gather(idxb, [gi]))
            v1_l.append(plsc.load_gather(w1b, [gi]))
            v2_l.append(plsc.load_gather(w2b, [gi]))
            cols_l.append(fv)

    units = [(rb0 + t, vals_l, s_hbm)
             for t in range(4)
             for vals_l, s_hbm in ((v1_l, s1_hbm), (v2_l, s2_hbm))]
    pending = [None, None]
    for u, (rb, vals_l, s_hbm) in enumerate(units):
        strip, sem = strips.at[u % 2], sems.at[u % 2]
        if pending[u % 2] is not None:
            pending[u % 2].wait()

        def _zero(i, carry):
            for uu in range(_L):
                fl = uu * _L + lanes
                r = i * (_L * _L // _CB) + jnp.right_shift(fl, 7)
                c = jnp.bitwise_and(fl, _CB - 1)
                plsc.store_scatter(strip, [r, c], zeros16)
            return carry
        lax.fori_loop(0, _RB * _CB // (_L * _L), _zero, 0)
        rbase = rb * _RB
        for rows, fv, vals in zip(rows_l, cols_l, vals_l):
            rl = rows - rbase
            mask = (rows >= rbase) & (rl < _RB)
            plsc.addupdate_scatter(strip, [rl, fv], vals, mask=mask)
        pending[u % 2] = pltpu.async_copy(
            strip, s_hbm.at[pl.ds(rbase, _RB), pl.ds(cbase, _CB)], sem)
    for p in pending:
        p.wait()


@jax.jit
def _densify_sc(idx, w1, w2):
    mesh = plsc.VectorSubcoreMesh(core_axis_name="c", subcore_axis_name="s")
    return pl.kernel(
        _densify_sc_body,
        out_type=[jax.ShapeDtypeStruct((_F, _F), jnp.float32),
                  jax.ShapeDtypeStruct((_F, _F), jnp.float32)],
        mesh=mesh,
        compiler_params=pltpu.CompilerParams(needs_layout_passes=False),
        scratch_types=[
            pltpu.VMEM((2, _RB, _CB), jnp.float32),
            pltpu.VMEM((_CB * _K,), jnp.int32),
            pltpu.VMEM((_CB * _K,), jnp.float32),
            pltpu.VMEM((_CB * _K,), jnp.float32),
            pltpu.SemaphoreType.DMA((2,)),
        ],
    )(idx.reshape(-1), w1.reshape(-1), w2.reshape(-1))


def _block_body(x_ref, s1_ref, s2_ref, b1_ref, b2_ref, g_ref, alpha_ref,
                out_ref):
    x = x_ref[...]
    ms = jnp.mean(x * x, axis=-1, keepdims=True)
    h = x * jax.lax.rsqrt(ms + _EPS) * g_ref[...]
    h1 = jnp.dot(h.astype(jnp.bfloat16), s1_ref[...].astype(jnp.bfloat16),
                 preferred_element_type=jnp.float32)
    h1 = jnp.maximum(h1 + b1_ref[...], 0.0)
    h2 = jnp.dot(h1.astype(jnp.bfloat16), s2_ref[...].astype(jnp.bfloat16),
                 preferred_element_type=jnp.float32)
    out_ref[...] = x + alpha_ref[0] * (h2 + b2_ref[...])


@functools.partial(jax.jit, static_argnames=("tb",))
def _residual_block(x, s1, s2, b1, b2, g, alpha, tb=512):
    t, f = x.shape
    grid = (t // tb,)
    return pl.pallas_call(
        _block_body,
        grid=grid,
        in_specs=[
            pl.BlockSpec((tb, f), lambda i: (i, 0)),
            pl.BlockSpec((f, f), lambda i: (0, 0)),
            pl.BlockSpec((f, f), lambda i: (0, 0)),
            pl.BlockSpec((1, f), lambda i: (0, 0)),
            pl.BlockSpec((1, f), lambda i: (0, 0)),
            pl.BlockSpec((1, f), lambda i: (0, 0)),
            pl.BlockSpec(memory_space=pltpu.SMEM),
        ],
        out_specs=pl.BlockSpec((tb, f), lambda i: (i, 0)),
        out_shape=jax.ShapeDtypeStruct((t, f), jnp.float32),
    )(x, s1, s2, b1, b2, g, alpha)


def kernel(x, in_index_per_out, w1, b1, w2, b2, norm_weight, alpha):
    s1, s2 = _densify_sc(in_index_per_out, w1, w2)
    return _residual_block(
        x, s1, s2,
        b1.reshape(1, -1), b2.reshape(1, -1), norm_weight.reshape(1, -1),
        alpha,
    )


# speedup vs baseline: 1.0023x; 1.0023x over previous
import functools

import jax
import jax.numpy as jnp
from jax import lax
from jax.experimental import pallas as pl
from jax.experimental.pallas import tpu as pltpu
from jax.experimental.pallas import tpu_sc as plsc

_EPS = 1e-6
_F, _K = 2048, 8
_NC, _NS, _L = 2, 16, 16
_NW = _NC * _NS
_CB = 128
_RB = 256


def _densify_sc_body(idx_hbm, w1_hbm, w2_hbm, s1_hbm, s2_hbm,
                     strips, idxb, w1b, w2b, sems):
    wid = lax.axis_index("s") * _NC + lax.axis_index("c")
    cb = jnp.right_shift(wid, 1)
    rb0 = jnp.bitwise_and(wid, 1) * 4
    cbase = cb * _CB
    lanes = lax.iota(jnp.int32, _L)
    zeros16 = jnp.zeros((_L,), jnp.float32)

    pltpu.sync_copy(idx_hbm.at[pl.ds(cbase * _K, _CB * _K)], idxb)
    pltpu.sync_copy(w1_hbm.at[pl.ds(cbase * _K, _CB * _K)], w1b)
    pltpu.sync_copy(w2_hbm.at[pl.ds(cbase * _K, _CB * _K)], w2b)
    rows_l, cols_l, v1_l, v2_l = [], [], [], []
    for k in range(_K):
        for half in range(_CB // _L):
            fv = lanes + half * _L
            gi = fv * _K + k
            rows_l.append(plsc.load_gather(idxb, [gi]))
            v1_l.append(plsc.load_gather(w1b, [gi]))
            v2_l.append(plsc.load_gather(w2b, [gi]))
            cols_l.append(fv)

    units = [(rb0 + t, vals_l, s_hbm)
             for t in range(4)
             for vals_l, s_hbm in ((v1_l, s1_hbm), (v2_l, s2_hbm))]
    pending = [None, None]
    for u, (rb, vals_l, s_hbm) in enumerate(units):
        strip, sem = strips.at[u % 2], sems.at[u % 2]
        if pending[u % 2] is not None:
            pending[u % 2].wait()

        def _zero(i, carry):
            for uu in range(_L):
                fl = uu * _L + lanes
                r = i * (_L * _L // _CB) + jnp.right_shift(fl, 7)
                c = jnp.bitwise_and(fl, _CB - 1)
                plsc.store_scatter(strip, [r, c], zeros16)
            return carry
        lax.fori_loop(0, _RB * _CB // (_L * _L), _zero, 0)
        rbase = rb * _RB
        for rows, fv, vals in zip(rows_l, cols_l, vals_l):
            rl = rows - rbase
            mask = (rows >= rbase) & (rl < _RB)
            plsc.addupdate_scatter(strip, [rl, fv], vals, mask=mask)
        pending[u % 2] = pltpu.async_copy(
            strip, s_hbm.at[pl.ds(rbase, _RB), pl.ds(cbase, _CB)], sem)
    for p in pending:
        p.wait()


@jax.jit
def _densify_sc(idx, w1, w2):
    mesh = plsc.VectorSubcoreMesh(core_axis_name="c", subcore_axis_name="s")
    return pl.kernel(
        _densify_sc_body,
        out_type=[jax.ShapeDtypeStruct((_F, _F), jnp.float32),
                  jax.ShapeDtypeStruct((_F, _F), jnp.float32)],
        mesh=mesh,
        compiler_params=pltpu.CompilerParams(needs_layout_passes=False),
        scratch_types=[
            pltpu.VMEM((2, _RB, _CB), jnp.float32),
            pltpu.VMEM((_CB * _K,), jnp.int32),
            pltpu.VMEM((_CB * _K,), jnp.float32),
            pltpu.VMEM((_CB * _K,), jnp.float32),
            pltpu.SemaphoreType.DMA((2,)),
        ],
    )(idx.reshape(-1), w1.reshape(-1), w2.reshape(-1))


def _block_body(x_ref, s1_ref, s2_ref, b1_ref, b2_ref, g_ref, alpha_ref,
                out_ref):
    x = x_ref[...]
    ms = jnp.mean(x * x, axis=-1, keepdims=True)
    h = x * jax.lax.rsqrt(ms + _EPS) * g_ref[...]
    h1 = jnp.dot(h, s1_ref[...], preferred_element_type=jnp.float32)
    h1 = jnp.maximum(h1 + b1_ref[...], 0.0)
    h2 = jnp.dot(h1, s2_ref[...], preferred_element_type=jnp.float32)
    out_ref[...] = x + alpha_ref[0] * (h2 + b2_ref[...])


@functools.partial(jax.jit, static_argnames=("tb",))
def _residual_block(x, s1, s2, b1, b2, g, alpha, tb=512):
    t, f = x.shape
    grid = (t // tb,)
    return pl.pallas_call(
        _block_body,
        grid=grid,
        in_specs=[
            pl.BlockSpec((tb, f), lambda i: (i, 0)),
            pl.BlockSpec((f, f), lambda i: (0, 0)),
            pl.BlockSpec((f, f), lambda i: (0, 0)),
            pl.BlockSpec((1, f), lambda i: (0, 0)),
            pl.BlockSpec((1, f), lambda i: (0, 0)),
            pl.BlockSpec((1, f), lambda i: (0, 0)),
            pl.BlockSpec(memory_space=pltpu.SMEM),
        ],
        out_specs=pl.BlockSpec((tb, f), lambda i: (i, 0)),
        out_shape=jax.ShapeDtypeStruct((t, f), jnp.float32),
    )(x, s1, s2, b1, b2, g, alpha)


def kernel(x, in_index_per_out, w1, b1, w2, b2, norm_weight, alpha):
    s1, s2 = _densify_sc(in_index_per_out, w1, w2)
    return _residual_block(
        x, s1, s2,
        b1.reshape(1, -1), b2.reshape(1, -1), norm_weight.reshape(1, -1),
        alpha,
    )
